# Initial kernel scaffold; baseline (speedup 1.0000x reference)
import functools

import jax
import jax.numpy as jnp
from jax import lax
from jax.experimental import pallas as pl
from jax.experimental.pallas import tpu as pltpu
from jax.experimental.pallas import tpu_sc as plsc

_N_IN = 100000
_N6 = 40000
_N7 = 10000
_NT = _N6 + _N7
_D = 128

_NW = 32
_B = 40
_NB6 = _N6 // _B
_NB7 = _N7 // _B
_NBT = _NT // _B

_mesh = plsc.VectorSubcoreMesh(core_axis_name="c", subcore_axis_name="s")


def _wid():
    return lax.axis_index("s") * 2 + lax.axis_index("c")


@functools.partial(
    pl.kernel,
    out_type=jax.ShapeDtypeStruct((_NT, _D), jnp.float32),
    mesh=_mesh,
    scratch_types=[
        pltpu.VMEM((_B * 6,), jnp.int32),
        pltpu.VMEM((_B * 7,), jnp.int32),
        pltpu.VMEM((_B * 6, _D), jnp.float32),
        pltpu.VMEM((_B * 7, _D), jnp.float32),
        pltpu.VMEM((_B, 6), jnp.float32),
        pltpu.VMEM((_B, 7), jnp.float32),
        pltpu.VMEM((_B, _D), jnp.float32),
        pltpu.SemaphoreType.DMA,
    ],
)
def _pool(x_hbm, w6_hbm, w7_hbm, i6_hbm, i7_hbm, pooled_hbm,
          idx6_v, idx7_v, rows6_v, rows7_v, w6_v, w7_v, out_v, sem):
    wid = _wid()

    def block(deg, t0, idx_v, rows_v, w_v, i_hbm, w_hbm, out0):
        pltpu.sync_copy(i_hbm.at[pl.ds(t0 * deg, _B * deg)], idx_v)
        pltpu.sync_copy(w_hbm.at[pl.ds(t0, _B)], w_v)
        pltpu.async_copy(x_hbm.at[idx_v], rows_v, sem).wait()

        @pl.loop(0, _B)
        def _(r):
            base = r * deg
            w = [w_v[r, k] for k in range(deg)]
            for c in range(_D // 16):
                sl = pl.ds(c * 16, 16)
                acc = w[0] * rows_v[base, sl]
                for k in range(1, deg):
                    acc = acc + w[k] * rows_v[base + k, sl]
            out_v[r, sl] = acc

        pltpu.sync_copy(out_v, pooled_hbm.at[pl.ds(out0 + t0, _B)])

    @pl.loop(0, (_NB6 + _NW - 1) // _NW)
    def _(it):
        bid = wid + it * _NW

        @pl.when(bid < _NB6)
        def _():
            block(6, bid * _B, idx6_v, rows6_v, w6_v, i6_hbm, w6_hbm, 0)

    @pl.loop(0, (_NB7 + _NW - 1) // _NW)
    def _(it):
        bid = wid + it * _NW

        @pl.when(bid < _NB7)
        def _():
            block(7, bid * _B, idx7_v, rows7_v, w7_v, i7_hbm, w7_hbm, _N6)


@functools.partial(
    pl.kernel,
    out_type=jax.ShapeDtypeStruct((_NT, _D), jnp.float32),
    mesh=_mesh,
    scratch_types=[
        pltpu.VMEM((_B,), jnp.int32),
        pltpu.VMEM((_B, _D), jnp.float32),
        pltpu.SemaphoreType.DMA,
    ],
)
def _reorder(pooled_hbm, tgt_hbm, out_hbm, tidx_v, rows_v, sem):
    wid = _wid()

    @pl.loop(0, (_NBT + _NW - 1) // _NW)
    def _(it):
        bid = wid + it * _NW

        @pl.when(bid < _NBT)
        def _():
            p0 = bid * _B
            pltpu.sync_copy(tgt_hbm.at[pl.ds(p0, _B)], tidx_v)
            pltpu.async_copy(pooled_hbm.at[tidx_v], rows_v, sem).wait()
            pltpu.sync_copy(rows_v, out_hbm.at[pl.ds(p0, _B)])


def kernel(x_in, weights_6, weights_7, indices_6, indices_7, indices_target):
    pooled = _pool(x_in, weights_6, weights_7, indices_6, indices_7)
    return _reorder(pooled, indices_target)


# speedup vs baseline: 2.5754x; 2.5754x over previous
import functools

import jax
import jax.numpy as jnp
from jax import lax
from jax.experimental import pallas as pl
from jax.experimental.pallas import tpu as pltpu
from jax.experimental.pallas import tpu_sc as plsc

_N_IN = 100000
_N6 = 40000
_N7 = 10000
_NT = _N6 + _N7
_D = 128

_NW = 32
_B = 40
_NB6 = _N6 // _B
_NB7 = _N7 // _B
_NBT = _NT // _B

_mesh = plsc.VectorSubcoreMesh(core_axis_name="c", subcore_axis_name="s")


def _wid():
    return lax.axis_index("s") * 2 + lax.axis_index("c")


@functools.partial(
    pl.kernel,
    out_type=jax.ShapeDtypeStruct((_NT, _D), jnp.float32),
    mesh=_mesh,
    scratch_types=[
        pltpu.VMEM((_B * 6,), jnp.int32),
        pltpu.VMEM((_B * 7,), jnp.int32),
        pltpu.VMEM((_B * 6, _D), jnp.float32),
        pltpu.VMEM((_B * 7, _D), jnp.float32),
        pltpu.VMEM((_B * 16,), jnp.float32),
        pltpu.VMEM((_B, _D), jnp.float32),
        pltpu.SemaphoreType.DMA,
    ],
)
def _pool(x_hbm, w6_hbm, w7_hbm, i6_hbm, i7_hbm, pooled_hbm,
          idx6_v, idx7_v, rows6_v, rows7_v, w_v, out_v, sem):
    wid = _wid()

    def block(deg, t0, idx_v, rows_v, i_hbm, w_hbm, out0):
        pltpu.sync_copy(i_hbm.at[pl.ds(t0 * deg, _B * deg)], idx_v)
        pltpu.sync_copy(w_hbm.at[pl.ds(t0 * 16, _B * 16)], w_v)
        pltpu.async_copy(x_hbm.at[idx_v], rows_v, sem).wait()

        @pl.loop(0, _B)
        def _(r):
            base = r * deg
            wrow = w_v[pl.ds(pl.multiple_of(r * 16, 16), 16)]
            for c in range(_D // 16):
                sl = pl.ds(c * 16, 16)
                acc = wrow[0] * rows_v[base, sl]
                for k in range(1, deg):
                    acc = acc + wrow[k] * rows_v[base + k, sl]
                out_v[r, sl] = acc

        pltpu.sync_copy(out_v, pooled_hbm.at[pl.ds(out0 + t0, _B)])

    @pl.loop(0, (_NB6 + _NW - 1) // _NW)
    def _(it):
        bid = wid + it * _NW

        @pl.when(bid < _NB6)
        def _():
            block(6, bid * _B, idx6_v, rows6_v, i6_hbm, w6_hbm, 0)

    @pl.loop(0, (_NB7 + _NW - 1) // _NW)
    def _(it):
        bid = wid + it * _NW

        @pl.when(bid < _NB7)
        def _():
            block(7, bid * _B, idx7_v, rows7_v, i7_hbm, w7_hbm, _N6)


@functools.partial(
    pl.kernel,
    out_type=jax.ShapeDtypeStruct((_NT, _D), jnp.float32),
    mesh=_mesh,
    scratch_types=[
        pltpu.VMEM((_B,), jnp.int32),
        pltpu.VMEM((_B, _D), jnp.float32),
        pltpu.SemaphoreType.DMA,
    ],
)
def _reorder(pooled_hbm, tgt_hbm, out_hbm, tidx_v, rows_v, sem):
    wid = _wid()

    @pl.loop(0, (_NBT + _NW - 1) // _NW)
    def _(it):
        bid = wid + it * _NW

        @pl.when(bid < _NBT)
        def _():
            p0 = bid * _B
            pltpu.sync_copy(tgt_hbm.at[pl.ds(p0, _B)], tidx_v)
            pltpu.async_copy(pooled_hbm.at[tidx_v], rows_v, sem).wait()
            pltpu.sync_copy(rows_v, out_hbm.at[pl.ds(p0, _B)])


def kernel(x_in, weights_6, weights_7, indices_6, indices_7, indices_target):
    w6p = jnp.zeros((_N6, 16), jnp.float32).at[:, :6].set(weights_6)
    w7p = jnp.zeros((_N7, 16), jnp.float32).at[:, :7].set(weights_7)
    pooled = _pool(x_in, w6p.reshape(-1), w7p.reshape(-1),
                   indices_6, indices_7)
    return _reorder(pooled, indices_target)


# speedup vs baseline: 3.6510x; 1.4176x over previous
import functools

import jax
import jax.numpy as jnp
from jax import lax
from jax.experimental import pallas as pl
from jax.experimental.pallas import tpu as pltpu
from jax.experimental.pallas import tpu_sc as plsc

_N_IN = 100000
_N6 = 40000
_N7 = 10000
_NT = _N6 + _N7
_D = 128

_NW = 32
_B = 40
_NB6 = _N6 // _B
_NB7 = _N7 // _B
_NBA = _NB6 + _NB7
_ITA = (_NBA + _NW - 1) // _NW

_BT = 200
_NBT = _NT // _BT
_ITB = (_NBT + _NW - 1) // _NW

_mesh = plsc.VectorSubcoreMesh(core_axis_name="c", subcore_axis_name="s")


def _wid():
    return lax.axis_index("s") * 2 + lax.axis_index("c")


@functools.partial(
    pl.kernel,
    out_type=jax.ShapeDtypeStruct((_NT, _D), jnp.float32),
    mesh=_mesh,
    scratch_types=[
        pltpu.VMEM((_B * 7,), jnp.int32),
        pltpu.VMEM((_B * 7,), jnp.int32),
        pltpu.VMEM((_B * 7, _D), jnp.float32),
        pltpu.VMEM((_B * 7, _D), jnp.float32),
        pltpu.VMEM((_B * 16,), jnp.float32),
        pltpu.VMEM((_B * 16,), jnp.float32),
        pltpu.VMEM((_B, _D), jnp.float32),
        pltpu.VMEM((_B, _D), jnp.float32),
        pltpu.SemaphoreType.DMA,
        pltpu.SemaphoreType.DMA,
        pltpu.SemaphoreType.DMA,
        pltpu.SemaphoreType.DMA,
    ],
)
def _pool(x_hbm, w6_hbm, w7_hbm, i6_hbm, i7_hbm, pooled_hbm,
          idx0, idx1, rows0, rows1, w0, w1, outv0, outv1, g0, g1, o0, o1):
    wid = _wid()
    idx = (idx0, idx1)
    rows = (rows0, rows1)
    wv = (w0, w1)
    outv = (outv0, outv1)
    gsem = (g0, g1)
    osem = (o0, o1)

    def stage(it, s):
        bid = wid + it * _NW

        @pl.when(bid < _NB6)
        def _():
            t0 = bid * _B
            pltpu.sync_copy(i6_hbm.at[pl.ds(t0 * 6, _B * 6)],
                            idx[s].at[pl.ds(0, _B * 6)])
            pltpu.sync_copy(w6_hbm.at[pl.ds(t0 * 16, _B * 16)], wv[s])
            pltpu.async_copy(x_hbm.at[idx[s].at[pl.ds(0, _B * 6)]],
                             rows[s].at[pl.ds(0, _B * 6)], gsem[s])

        @pl.when(jnp.logical_and(bid >= _NB6, bid < _NBA))
        def _():
            t0 = (bid - _NB6) * _B
            pltpu.sync_copy(i7_hbm.at[pl.ds(t0 * 7, _B * 7)], idx[s])
            pltpu.sync_copy(w7_hbm.at[pl.ds(t0 * 16, _B * 16)], wv[s])
            pltpu.async_copy(x_hbm.at[idx[s]], rows[s], gsem[s])

    def compute(it, s):
        bid = wid + it * _NW

        def body(deg, out0, t0):
            pltpu.make_async_copy(
                x_hbm.at[pl.ds(0, _B * deg)],
                rows[s].at[pl.ds(0, _B * deg)], gsem[s]).wait()

            @pl.loop(0, _B)
            def _(r):
                base = r * deg
                wrow = wv[s][pl.ds(pl.multiple_of(r * 16, 16), 16)]
                for c in range(_D // 16):
                    sl = pl.ds(c * 16, 16)
                    acc = wrow[0] * rows[s][base, sl]
                    for k in range(1, deg):
                        acc = acc + wrow[k] * rows[s][base + k, sl]
                    outv[s][r, sl] = acc

            pltpu.async_copy(outv[s], pooled_hbm.at[pl.ds(out0 + t0, _B)],
                             osem[s])

        @pl.when(bid < _NB6)
        def _():
            body(6, 0, bid * _B)

        @pl.when(jnp.logical_and(bid >= _NB6, bid < _NBA))
        def _():
            body(7, _N6, (bid - _NB6) * _B)

    def drain_store(s):
        pltpu.make_async_copy(outv[s], pooled_hbm.at[pl.ds(0, _B)],
                              osem[s]).wait()

    stage(0, 0)

    @pl.loop(0, _ITA, step=2)
    def _(it0):
        for b in range(2):
            it = it0 + b
            ns = 1 - b

            @pl.when(it + 1 < _ITA)
            def _():
                pl.when(it >= 1)(lambda: drain_store(ns))
                stage(it + 1, ns)

            compute(it, b)

    drain_store(0)
    pl.when(wid + (_ITA - 1) * _NW < _NBA)(lambda: drain_store(1))


@functools.partial(
    pl.kernel,
    out_type=jax.ShapeDtypeStruct((_NT, _D), jnp.float32),
    mesh=_mesh,
    scratch_types=[
        pltpu.VMEM((_BT,), jnp.int32),
        pltpu.VMEM((_BT,), jnp.int32),
        pltpu.VMEM((_BT, _D), jnp.float32),
        pltpu.VMEM((_BT, _D), jnp.float32),
        pltpu.SemaphoreType.DMA,
        pltpu.SemaphoreType.DMA,
        pltpu.SemaphoreType.DMA,
        pltpu.SemaphoreType.DMA,
    ],
)
def _reorder(pooled_hbm, tgt_hbm, out_hbm,
             tidx0, tidx1, rows0, rows1, g0, g1, o0, o1):
    wid = _wid()
    tidx = (tidx0, tidx1)
    rows = (rows0, rows1)
    gsem = (g0, g1)
    osem = (o0, o1)

    def stage(it, s):
        bid = wid + it * _NW

        @pl.when(bid < _NBT)
        def _():
            pltpu.sync_copy(tgt_hbm.at[pl.ds(bid * _BT, _BT)], tidx[s])
            pltpu.async_copy(pooled_hbm.at[tidx[s]], rows[s], gsem[s])

    def flush(it, s):
        bid = wid + it * _NW

        @pl.when(bid < _NBT)
        def _():
            pltpu.make_async_copy(pooled_hbm.at[pl.ds(0, _BT)], rows[s],
                                  gsem[s]).wait()
            pltpu.async_copy(rows[s], out_hbm.at[pl.ds(bid * _BT, _BT)],
                             osem[s])

    def drain_store(s):
        pltpu.make_async_copy(rows[s], out_hbm.at[pl.ds(0, _BT)],
                              osem[s]).wait()

    stage(0, 0)

    @pl.loop(0, _ITB, step=2)
    def _(it0):
        for b in range(2):
            it = it0 + b
            ns = 1 - b

            @pl.when(it + 1 < _ITB)
            def _():
                pl.when(it >= 1)(lambda: drain_store(ns))
                stage(it + 1, ns)

            flush(it, b)

    pl.when(wid + (_ITB - 2) * _NW < _NBT)(lambda: drain_store(0))
    pl.when(wid + (_ITB - 1) * _NW < _NBT)(lambda: drain_store(1))


def kernel(x_in, weights_6, weights_7, indices_6, indices_7, indices_target):
    w6p = jnp.zeros((_N6, 16), jnp.float32).at[:, :6].set(weights_6)
    w7p = jnp.zeros((_N7, 16), jnp.float32).at[:, :7].set(weights_7)
    pooled = _pool(x_in, w6p.reshape(-1), w7p.reshape(-1),
                   indices_6, indices_7)
    return _reorder(pooled, indices_target)
